# baseline (speedup 1.0000x reference)
import functools

import jax
import jax.numpy as jnp
from jax import lax
from jax.experimental import pallas as pl
from jax.experimental.pallas import tpu as pltpu
from jax.experimental.pallas import tpu_sc as plsc

_CHUNK = 16
_NBUF = 4


def _make_sc_kernel(B, S, H, dtype):
    info = plsc.get_sparse_core_info()
    nw = info.num_cores * info.num_subcores
    spw = S // nw
    nch = spw // _CHUNK
    assert B == _NBUF and nch % 2 == 0 and nch >= 4
    mesh = plsc.VectorSubcoreMesh(core_axis_name="c", subcore_axis_name="s")

    @functools.partial(
        pl.kernel,
        out_type=jax.ShapeDtypeStruct((B * S, H), dtype),
        mesh=mesh,
        scratch_types=[
            [pltpu.VMEM((_CHUNK, H), dtype)] * _NBUF,
            [pltpu.VMEM((_CHUNK, H), dtype)] * 2,
            [pltpu.SemaphoreType.DMA] * _NBUF,
            [pltpu.SemaphoreType.DMA] * _NBUF,
            [pltpu.SemaphoreType.DMA] * 2,
        ],
    )
    def sc_add(x_hbm, pe_hbm, out_hbm, xbufs, pebufs, sins, souts, spes):
        wid = lax.axis_index("s") * info.num_cores + lax.axis_index("c")
        s0 = wid * spw

        def xrow(c, b):
            return pl.multiple_of(b * S + s0 + c * _CHUNK, _CHUNK)

        def issue_in(c, b, slot):
            pltpu.async_copy(
                x_hbm.at[pl.ds(xrow(c, b), _CHUNK)], xbufs[slot], sins[slot]
            )

        def issue_out(c, b, slot):
            pltpu.async_copy(
                xbufs[slot], out_hbm.at[pl.ds(xrow(c, b), _CHUNK)], souts[slot]
            )

        def issue_pe(c, q):
            row = pl.multiple_of(s0 + c * _CHUNK, _CHUNK)
            pltpu.async_copy(pe_hbm.at[pl.ds(row, _CHUNK)], pebufs[q], spes[q])

        def wait_in(slot):
            pltpu.make_async_copy(
                x_hbm.at[pl.ds(0, _CHUNK)], xbufs[slot], sins[slot]
            ).wait()

        def wait_out(slot):
            pltpu.make_async_copy(
                xbufs[slot], out_hbm.at[pl.ds(0, _CHUNK)], souts[slot]
            ).wait()

        def wait_pe(q):
            pltpu.make_async_copy(
                pe_hbm.at[pl.ds(0, _CHUNK)], pebufs[q], spes[q]
            ).wait()

        def compute(slot, q):
            xb = xbufs[slot]
            pb = pebufs[q]

            @plsc.parallel_loop(0, _CHUNK, 1)
            def _(r):
                @plsc.parallel_loop(0, H, 16, unroll=16)
                def _(col):
                    plsc.addupdate(xb.at[r, pl.ds(col, 16)], pb[r, pl.ds(col, 16)])

        def phase(c, b, q, ring=True):
            rs = (b + 1) % _NBUF
            if ring:
                wait_out(rs)
                if b < _NBUF - 1:
                    issue_in(c, b + 1, rs)
                else:
                    issue_in(c + 1, 0, rs)
            wait_in(b)
            compute(b, q)
            issue_out(c, b, b)

        issue_pe(0, 0)
        for b in range(_NBUF):
            issue_in(0, b, b)
        wait_pe(0)
        issue_pe(1, 1)
        for b in range(_NBUF - 1):
            phase(0, b, 0, ring=False)
        phase(0, _NBUF - 1, 0)

        @pl.loop(0, (nch - 2) // 2)
        def _(g):
            c1 = pl.multiple_of(2 * g + 1, 1)
            c2 = pl.multiple_of(2 * g + 2, 1)
            wait_pe(1)
            issue_pe(c2, 0)
            for b in range(_NBUF):
                phase(c1, b, 1)
            wait_pe(0)
            issue_pe(c2 + 1, 1)
            for b in range(_NBUF):
                phase(c2, b, 0)

        wait_pe(1)
        for b in range(_NBUF - 1):
            phase(nch - 1, b, 1)
        wait_in(_NBUF - 1)
        compute(_NBUF - 1, 1)
        issue_out(nch - 1, _NBUF - 1, _NBUF - 1)
        for slot in range(_NBUF):
            wait_out(slot)

    return sc_add


def kernel(x, pe_table):
    B, S, H = x.shape
    sc_add = _make_sc_kernel(B, S, H, x.dtype)
    out = sc_add(x.reshape(B * S, H), pe_table)
    return out.reshape(B, S, H)


# speedup vs baseline: 1.0061x; 1.0061x over previous
import functools

import jax
import jax.numpy as jnp
from jax import lax
from jax.experimental import pallas as pl
from jax.experimental.pallas import tpu as pltpu
from jax.experimental.pallas import tpu_sc as plsc

_CHUNK = 16
_NBUF = 4


def _make_sc_kernel(B, S, H, dtype):
    info = plsc.get_sparse_core_info()
    nw = info.num_cores * info.num_subcores
    spw = S // nw
    nch = spw // _CHUNK
    assert B == _NBUF and nch % 2 == 0 and nch >= 4
    mesh = plsc.VectorSubcoreMesh(core_axis_name="c", subcore_axis_name="s")

    @functools.partial(
        pl.kernel,
        out_type=jax.ShapeDtypeStruct((B * S, H), dtype),
        mesh=mesh,
        scratch_types=[
            [pltpu.VMEM((_CHUNK, H), dtype)] * _NBUF,
            [pltpu.VMEM((_CHUNK, H), dtype)] * 2,
            [pltpu.SemaphoreType.DMA] * _NBUF,
            [pltpu.SemaphoreType.DMA] * _NBUF,
            [pltpu.SemaphoreType.DMA] * 2,
        ],
    )
    def sc_add(x_hbm, pe_hbm, out_hbm, xbufs, pebufs, sins, souts, spes):
        wid = lax.axis_index("s") * info.num_cores + lax.axis_index("c")
        s0 = wid * spw

        def xrow(c, b):
            return pl.multiple_of(b * S + s0 + c * _CHUNK, _CHUNK)

        def issue_in(c, b, slot):
            pltpu.async_copy(
                x_hbm.at[pl.ds(xrow(c, b), _CHUNK)], xbufs[slot], sins[slot]
            )

        def issue_out(c, b, slot):
            pltpu.async_copy(
                xbufs[slot], out_hbm.at[pl.ds(xrow(c, b), _CHUNK)], souts[slot]
            )

        def issue_pe(c, q):
            row = pl.multiple_of(s0 + c * _CHUNK, _CHUNK)
            pltpu.async_copy(pe_hbm.at[pl.ds(row, _CHUNK)], pebufs[q], spes[q])

        def wait_in(slot):
            pltpu.make_async_copy(
                x_hbm.at[pl.ds(0, _CHUNK)], xbufs[slot], sins[slot]
            ).wait()

        def wait_out(slot):
            pltpu.make_async_copy(
                xbufs[slot], out_hbm.at[pl.ds(0, _CHUNK)], souts[slot]
            ).wait()

        def wait_pe(q):
            pltpu.make_async_copy(
                pe_hbm.at[pl.ds(0, _CHUNK)], pebufs[q], spes[q]
            ).wait()

        def compute_half(slot, q, h):
            xb = xbufs[slot]
            pb = pebufs[q]
            hr = _CHUNK // 2

            @plsc.parallel_loop(h * hr, (h + 1) * hr, 1)
            def _(r):
                @plsc.parallel_loop(0, H, 16, unroll=8)
                def _(col):
                    plsc.addupdate(xb.at[r, pl.ds(col, 16)], pb[r, pl.ds(col, 16)])

        def issue_out_half(c, b, slot, h):
            hr = _CHUNK // 2
            pltpu.async_copy(
                xbufs[slot].at[pl.ds(h * hr, hr)],
                out_hbm.at[pl.ds(pl.multiple_of(xrow(c, b) + h * hr, hr), hr)],
                souts[slot],
            )

        def emit_compute_out(c, b, slot, q):
            compute_half(slot, q, 0)
            issue_out_half(c, b, slot, 0)
            compute_half(slot, q, 1)
            issue_out_half(c, b, slot, 1)

        def phase(c, b, q, ring=True):
            rs = (b + 1) % _NBUF
            if ring:
                wait_out(rs)
                if b < _NBUF - 1:
                    issue_in(c, b + 1, rs)
                else:
                    issue_in(c + 1, 0, rs)
            wait_in(b)
            emit_compute_out(c, b, b, q)

        issue_pe(0, 0)
        for b in range(_NBUF):
            issue_in(0, b, b)
        wait_pe(0)
        issue_pe(1, 1)
        for b in range(_NBUF - 1):
            phase(0, b, 0, ring=False)
        phase(0, _NBUF - 1, 0)

        @pl.loop(0, (nch - 2) // 2)
        def _(g):
            c1 = pl.multiple_of(2 * g + 1, 1)
            c2 = pl.multiple_of(2 * g + 2, 1)
            wait_pe(1)
            issue_pe(c2, 0)
            for b in range(_NBUF):
                phase(c1, b, 1)
            wait_pe(0)
            issue_pe(c2 + 1, 1)
            for b in range(_NBUF):
                phase(c2, b, 0)

        wait_pe(1)
        for b in range(_NBUF - 1):
            phase(nch - 1, b, 1)
        wait_in(_NBUF - 1)
        emit_compute_out(nch - 1, _NBUF - 1, _NBUF - 1, 1)
        for slot in range(_NBUF):
            wait_out(slot)

    return sc_add


def kernel(x, pe_table):
    B, S, H = x.shape
    sc_add = _make_sc_kernel(B, S, H, x.dtype)
    out = sc_add(x.reshape(B * S, H), pe_table)
    return out.reshape(B, S, H)
